# baseline (speedup 1.0000x reference)
import functools

import jax
import jax.numpy as jnp
from jax.experimental import pallas as pl
from jax.experimental.pallas import tpu as pltpu
from jax.experimental.pallas import tpu_sc as plsc

_HIDDEN = 64
_CHUNK = 256
_NW = 32
_RBLK = 8000
_MM_BLOCK = 8192


def _tc_repack(embedding):
    v, d = embedding.shape

    def repack_kernel(x_ref, o_ref):
        x = x_ref[...]
        o_ref[...] = jnp.concatenate([x, x], axis=1)

    return pl.pallas_call(
        repack_kernel,
        grid=(v // _RBLK,),
        in_specs=[pl.BlockSpec((_RBLK, d), lambda i: (i, 0))],
        out_specs=pl.BlockSpec((_RBLK, 2 * d), lambda i: (i, 0)),
        out_shape=jax.ShapeDtypeStruct((v, 2 * d), jnp.float32),
    )(embedding)


def _sc_gather(table_dup, idx_flat, d):
    n = idx_flat.shape[0]
    d2 = table_dup.shape[1]
    mesh = plsc.VectorSubcoreMesh(core_axis_name="core", subcore_axis_name="subcore")
    b_per_w = n // _NW
    n_chunks = b_per_w // _CHUNK

    @functools.partial(
        pl.kernel,
        out_type=jax.ShapeDtypeStruct((n, d2), jnp.float32),
        mesh=mesh,
        scratch_types=[
            pltpu.VMEM((b_per_w,), jnp.int32),
            pltpu.VMEM((_CHUNK, d2), jnp.float32),
            pltpu.VMEM((_CHUNK, d2), jnp.float32),
            pltpu.SemaphoreType.DMA,
            pltpu.SemaphoreType.DMA,
        ],
    )
    def gather_kernel(x_hbm, i_hbm, o_hbm, idx_v, rows_a, rows_b, sem_a, sem_b):
        wid = jax.lax.axis_index("subcore") * 2 + jax.lax.axis_index("core")
        base = wid * b_per_w
        pltpu.sync_copy(i_hbm.at[pl.ds(base, b_per_w)], idx_v)

        def gather_desc(c, rows, sem):
            return pltpu.make_async_copy(
                x_hbm.at[idx_v.at[pl.ds(c * _CHUNK, _CHUNK)]], rows, sem
            )

        def write_out(c, rows):
            pltpu.sync_copy(rows, o_hbm.at[pl.ds(base + c * _CHUNK, _CHUNK)])

        gather_desc(0, rows_a, sem_a).start()

        @pl.loop(0, n_chunks, step=2)
        def _(c):
            gather_desc(c + 1, rows_b, sem_b).start()
            gather_desc(c, rows_a, sem_a).wait()
            write_out(c, rows_a)

            @pl.when(c + 2 < n_chunks)
            def _():
                gather_desc(c + 2, rows_a, sem_a).start()

            gather_desc(c + 1, rows_b, sem_b).wait()
            write_out(c + 1, rows_b)

    return gather_kernel(table_dup, idx_flat)


def _tc_project(x, W, b):
    n, d2 = x.shape
    d = W.shape[0]

    def mm_kernel(x_ref, w_ref, b_ref, o_ref):
        o_ref[...] = (
            jnp.dot(
                x_ref[:, :d], w_ref[...], preferred_element_type=jnp.float32
            )
            + b_ref[...]
        )

    return pl.pallas_call(
        mm_kernel,
        grid=(n // _MM_BLOCK,),
        in_specs=[
            pl.BlockSpec((_MM_BLOCK, d2), lambda i: (i, 0)),
            pl.BlockSpec((d, d), lambda i: (0, 0)),
            pl.BlockSpec((1, d), lambda i: (0, 0)),
        ],
        out_specs=pl.BlockSpec((_MM_BLOCK, d), lambda i: (i, 0)),
        out_shape=jax.ShapeDtypeStruct((n, d), jnp.float32),
    )(x, W, b.reshape(1, d))


def kernel(input_ids, attention_mask, embedding, W, b):
    del attention_mask
    bsz, seqlen = input_ids.shape
    n = bsz * seqlen
    idx_flat = input_ids.reshape(n)
    table_dup = _tc_repack(embedding)
    gathered = _sc_gather(table_dup, idx_flat, embedding.shape[1])
    out = _tc_project(gathered, W, b)
    return out.reshape(bsz, seqlen, _HIDDEN)


# speedup vs baseline: 1.7061x; 1.7061x over previous
import functools

import jax
import jax.numpy as jnp
from jax.experimental import pallas as pl
from jax.experimental.pallas import tpu as pltpu
from jax.experimental.pallas import tpu_sc as plsc

_CHUNK = 256
_NW = 32
_RBLK = 8192
_LBLK = 2


def _tc_prep(emb_t):
    d, v = emb_t.shape

    def prep_kernel(x_ref, o_ref):
        xt = x_ref[...].T
        o_ref[...] = jnp.concatenate([xt, xt], axis=1)

    return pl.pallas_call(
        prep_kernel,
        grid=(pl.cdiv(v, _RBLK),),
        in_specs=[pl.BlockSpec((d, _RBLK), lambda i: (0, i))],
        out_specs=pl.BlockSpec((_RBLK, 2 * d), lambda i: (i, 0)),
        out_shape=jax.ShapeDtypeStruct((v, 2 * d), jnp.float32),
    )(emb_t)


def _sc_gather(table_dup, idx_flat):
    n = idx_flat.shape[0]
    d2 = table_dup.shape[1]
    mesh = plsc.VectorSubcoreMesh(core_axis_name="core", subcore_axis_name="subcore")
    b_per_w = n // _NW
    n_chunks = b_per_w // _CHUNK

    @functools.partial(
        pl.kernel,
        out_type=jax.ShapeDtypeStruct((n, d2), jnp.float32),
        mesh=mesh,
        scratch_types=[
            pltpu.VMEM((b_per_w,), jnp.int32),
            pltpu.VMEM((_CHUNK, d2), jnp.float32),
            pltpu.VMEM((_CHUNK, d2), jnp.float32),
            pltpu.SemaphoreType.DMA,
            pltpu.SemaphoreType.DMA,
        ],
    )
    def gather_kernel(x_hbm, i_hbm, o_hbm, idx_v, rows_a, rows_b, sem_a, sem_b):
        wid = jax.lax.axis_index("subcore") * 2 + jax.lax.axis_index("core")
        base = wid * b_per_w
        pltpu.sync_copy(i_hbm.at[pl.ds(base, b_per_w)], idx_v)

        def gather_desc(c, rows, sem):
            return pltpu.make_async_copy(
                x_hbm.at[idx_v.at[pl.ds(c * _CHUNK, _CHUNK)]], rows, sem
            )

        def write_out(c, rows):
            pltpu.sync_copy(rows, o_hbm.at[pl.ds(base + c * _CHUNK, _CHUNK)])

        gather_desc(0, rows_a, sem_a).start()

        @pl.loop(0, n_chunks, step=2)
        def _(c):
            gather_desc(c + 1, rows_b, sem_b).start()
            gather_desc(c, rows_a, sem_a).wait()
            write_out(c, rows_a)

            @pl.when(c + 2 < n_chunks)
            def _():
                gather_desc(c + 2, rows_a, sem_a).start()

            gather_desc(c + 1, rows_b, sem_b).wait()
            write_out(c + 1, rows_b)

    return gather_kernel(table_dup, idx_flat)


def _tc_project_t(g, W, b_col, seqlen, bsz):
    n, d2 = g.shape
    d = W.shape[0]
    rows_blk = _LBLK * bsz

    def mm_kernel(g_ref, w_ref, b_ref, o_ref):
        x = g_ref[:, :d]
        bias = b_ref[...]
        for j in range(_LBLK):
            xj = x[j * bsz : (j + 1) * bsz, :]
            yj = jax.lax.dot_general(
                w_ref[...],
                xj,
                (((0,), (1,)), ((), ())),
                preferred_element_type=jnp.float32,
            )
            o_ref[j] = yj + bias

    return pl.pallas_call(
        mm_kernel,
        grid=(n // rows_blk,),
        in_specs=[
            pl.BlockSpec((rows_blk, d2), lambda i: (i, 0)),
            pl.BlockSpec((d, d), lambda i: (0, 0)),
            pl.BlockSpec((d, 1), lambda i: (0, 0)),
        ],
        out_specs=pl.BlockSpec((_LBLK, d, bsz), lambda i: (i, 0, 0)),
        out_shape=jax.ShapeDtypeStruct((seqlen, d, bsz), jnp.float32),
    )(g, W, b_col)


def kernel(input_ids, attention_mask, embedding, W, b):
    del attention_mask
    bsz, seqlen = input_ids.shape
    idx_lmajor = input_ids.T.reshape(bsz * seqlen)
    emb_t = embedding.T
    table_dup = _tc_prep(emb_t)
    gathered = _sc_gather(table_dup, idx_lmajor)
    out_t = _tc_project_t(gathered, W, b.reshape(W.shape[0], 1), seqlen, bsz)
    return jnp.transpose(out_t, (2, 0, 1))
